# baseline (speedup 1.0000x reference)
import functools

import jax
import jax.numpy as jnp
from jax import lax
from jax.experimental import pallas as pl
from jax.experimental.pallas import tpu as pltpu
from jax.experimental.pallas import tpu_sc as plsc

USER_VOCAB = 1000000
ITEM_VOCAB = 1000000
EMB_DIM = 32
BATCH = 16384

NC = 2
NS = 16
NW = NC * NS
B_PER_W = BATCH // NW
CHUNK = 128
NCHUNK = B_PER_W // CHUNK
GROUPS = B_PER_W // 16


def _rsqrt(x):
    xi = lax.bitcast_convert_type(x, jnp.int32)
    yi = jnp.int32(0x5F3759DF) - (xi >> 1)
    y = lax.bitcast_convert_type(yi, jnp.float32)
    for _ in range(3):
        y = y * (1.5 - 0.5 * x * y * y)
    return y


@functools.partial(
    pl.kernel,
    out_type=jax.ShapeDtypeStruct((BATCH,), jnp.float32),
    mesh=plsc.VectorSubcoreMesh(core_axis_name="c", subcore_axis_name="s"),
    compiler_params=pltpu.CompilerParams(use_tc_tiling_on_sc=False),
    scratch_types=[
        pltpu.VMEM((NCHUNK, CHUNK), jnp.int32),
        pltpu.VMEM((NCHUNK, CHUNK), jnp.int32),
        pltpu.VMEM((B_PER_W, EMB_DIM), jnp.float32),
        pltpu.VMEM((B_PER_W, EMB_DIM), jnp.float32),
        pltpu.VMEM((B_PER_W,), jnp.float32),
        pltpu.SemaphoreType.DMA,
    ],
)
def _mf_sc_kernel(uid_hbm, iid_hbm, ut_hbm, it_hbm, out_hbm,
                  uidx, iidx, urows, irows, res, sem):
    wid = lax.axis_index("s") * NC + lax.axis_index("c")
    base = wid * B_PER_W

    pltpu.sync_copy(uid_hbm.at[pl.ds(wid * NCHUNK, NCHUNK)], uidx)
    pltpu.sync_copy(iid_hbm.at[pl.ds(wid * NCHUNK, NCHUNK)], iidx)

    copies = []
    for j in range(NCHUNK):
        copies.append(pltpu.async_copy(
            ut_hbm.at[uidx.at[j]], urows.at[pl.ds(j * CHUNK, CHUNK)], sem))
        copies.append(pltpu.async_copy(
            it_hbm.at[iidx.at[j]], irows.at[pl.ds(j * CHUNK, CHUNK)], sem))
    for c in copies:
        c.wait()

    lanes = lax.iota(jnp.int32, 16)
    last = jnp.full((16,), 15, jnp.int32)

    dnums = lax.GatherDimensionNumbers(
        offset_dims=(), collapsed_slice_dims=(0,), start_index_map=(0,))

    def shuffle(x, perm):
        return lax.gather(x, perm[:, None], dnums, (1,),
                          mode=lax.GatherScatterMode.PROMISE_IN_BOUNDS)

    def lane_sum(x):
        for k in (1, 2, 4, 8):
            x = x + shuffle(x, lanes ^ k)
        return x

    def group_body(g, carry):
        dvec = jnp.zeros((16,), jnp.float32)
        nvec = jnp.zeros((16,), jnp.float32)
        base_r = g * 16
        for j in range(16):
            r = base_r + j
            u0 = urows[r, 0:16]
            u1 = urows[r, 16:32]
            v0 = irows[r, 0:16]
            v1 = irows[r, 16:32]
            dot = lane_sum(u0 * v0 + u1 * v1)
            n2 = lane_sum(u0 * u0 + u1 * u1) * lane_sum(v0 * v0 + v1 * v1)
            m = lanes == j
            dvec = jnp.where(m, dot, dvec)
            nvec = jnp.where(m, n2, nvec)
        res[pl.ds(base_r, 16)] = 6.0 * dvec * _rsqrt(jnp.maximum(nvec, 1e-30))
        return carry

    lax.fori_loop(0, GROUPS, group_body, 0)

    pltpu.sync_copy(res, out_hbm.at[pl.ds(base, B_PER_W)])


def kernel(user_id, item_id, user_table, item_table):
    uid = user_id.astype(jnp.int32).reshape(NW * NCHUNK, CHUNK)
    iid = item_id.astype(jnp.int32).reshape(NW * NCHUNK, CHUNK)
    return _mf_sc_kernel(uid, iid, user_table, item_table)


# speedup vs baseline: 3.5753x; 3.5753x over previous
import functools

import jax
import jax.numpy as jnp
from jax import lax
from jax.experimental import pallas as pl
from jax.experimental.pallas import tpu as pltpu
from jax.experimental.pallas import tpu_sc as plsc

EMB_DIM = 32
BATCH = 16384

NC = 2
NS = 16
NW = NC * NS
B_PER_W = BATCH // NW
IDX_ROWS = B_PER_W // 128
WAVE = 16
NWAVE = B_PER_W // WAVE


@functools.partial(
    pl.kernel,
    out_type=jax.ShapeDtypeStruct((BATCH,), jnp.float32),
    mesh=plsc.VectorSubcoreMesh(core_axis_name="c", subcore_axis_name="s"),
    compiler_params=pltpu.CompilerParams(
        use_tc_tiling_on_sc=True,
        needs_layout_passes=False,
        disable_bounds_checks=True,
    ),
    scratch_types=[
        pltpu.VMEM((IDX_ROWS, 128), jnp.int32),
        pltpu.VMEM((IDX_ROWS, 128), jnp.int32),
        pltpu.VMEM((EMB_DIM, WAVE * 128), jnp.float32),
        pltpu.VMEM((EMB_DIM, B_PER_W), jnp.float32),
        pltpu.VMEM((EMB_DIM, B_PER_W), jnp.float32),
        pltpu.VMEM((B_PER_W,), jnp.float32),
        pltpu.SemaphoreType.DMA,
    ],
)
def _mf_sc_kernel(uid_hbm, iid_hbm, ut_hbm, it_hbm, out_hbm,
                  uidx, iidx, ring, ustage, istage, res, sem):
    wid = lax.axis_index("s") * NC + lax.axis_index("c")
    base = wid * B_PER_W

    pltpu.sync_copy(uid_hbm.at[pl.ds(wid * IDX_ROWS, IDX_ROWS)], uidx)
    pltpu.sync_copy(iid_hbm.at[pl.ds(wid * IDX_ROWS, IDX_ROWS)], iidx)

    lanes = lax.iota(jnp.int32, 16)

    def gather_pass(idx_ref, tab_hbm, stage):
        def wave_body(w, carry):
            idvec = idx_ref[w // 8, pl.ds((w % 8) * 16, 16)]
            copies = []
            for t in range(WAVE):
                idv = idvec[t]
                col = pl.multiple_of((idv // 128) * 128, 128)
                copies.append(pltpu.async_copy(
                    tab_hbm.at[:, pl.ds(col, 128)],
                    ring.at[:, pl.ds(t * 128, 128)], sem))
            for c in copies:
                c.wait()
            pos = lanes * 128 + (idvec % 128)
            for d in range(EMB_DIM):
                dv = jnp.full((16,), d, jnp.int32)
                stage[d, pl.ds(w * 16, 16)] = plsc.load_gather(ring, [dv, pos])
            return carry

        lax.fori_loop(0, NWAVE, wave_body, 0)

    gather_pass(uidx, ut_hbm, ustage)
    gather_pass(iidx, it_hbm, istage)

    def group_body(g, carry):
        dot = jnp.zeros((16,), jnp.float32)
        nu = jnp.zeros((16,), jnp.float32)
        nv = jnp.zeros((16,), jnp.float32)
        for d in range(EMB_DIM):
            u = ustage[d, pl.ds(g * 16, 16)]
            v = istage[d, pl.ds(g * 16, 16)]
            dot = dot + u * v
            nu = nu + u * u
            nv = nv + v * v
        x = jnp.maximum(nu * nv, 1e-30)
        xi = lax.bitcast_convert_type(x, jnp.int32)
        y = lax.bitcast_convert_type(
            jnp.int32(0x5F3759DF) - (xi >> 1), jnp.float32)
        for _ in range(3):
            y = y * (1.5 - 0.5 * x * y * y)
        res[pl.ds(g * 16, 16)] = 6.0 * dot * y
        return carry

    lax.fori_loop(0, B_PER_W // 16, group_body, 0)

    pltpu.sync_copy(res, out_hbm.at[pl.ds(base, B_PER_W)])


def kernel(user_id, item_id, user_table, item_table):
    uid = user_id.astype(jnp.int32).reshape(128, 128)
    iid = item_id.astype(jnp.int32).reshape(128, 128)
    return _mf_sc_kernel(uid, iid, user_table.T, item_table.T)


# speedup vs baseline: 3.8827x; 1.0860x over previous
import functools

import jax
import jax.numpy as jnp
from jax import lax
from jax.experimental import pallas as pl
from jax.experimental.pallas import tpu as pltpu
from jax.experimental.pallas import tpu_sc as plsc

EMB_DIM = 32
BATCH = 16384

NC = 2
NS = 16
NW = NC * NS
B_PER_W = BATCH // NW
IDX_ROWS = B_PER_W // 128
WIN = 8


@functools.partial(
    pl.kernel,
    out_type=jax.ShapeDtypeStruct((BATCH,), jnp.float32),
    mesh=plsc.VectorSubcoreMesh(core_axis_name="c", subcore_axis_name="s"),
    compiler_params=pltpu.CompilerParams(
        use_tc_tiling_on_sc=True,
        needs_layout_passes=False,
        disable_bounds_checks=True,
    ),
    scratch_types=(
        [
            pltpu.VMEM((IDX_ROWS, 128), jnp.int32),
            pltpu.VMEM((IDX_ROWS, 128), jnp.int32),
            pltpu.VMEM((EMB_DIM, WIN * 128), jnp.float32),
            pltpu.VMEM((EMB_DIM, WIN * 128), jnp.float32),
            pltpu.VMEM((EMB_DIM, B_PER_W), jnp.float32),
            pltpu.VMEM((EMB_DIM, B_PER_W), jnp.float32),
            pltpu.VMEM((B_PER_W,), jnp.float32),
        ]
        + [pltpu.SemaphoreType.DMA] * (2 * WIN)
    ),
)
def _mf_sc_kernel(uid_hbm, iid_hbm, ut_hbm, it_hbm, out_hbm,
                  uidx, iidx, uring, iring, ustage, istage, res, *sems):
    usems = sems[:WIN]
    isems = sems[WIN:]
    wid = lax.axis_index("s") * NC + lax.axis_index("c")
    base = wid * B_PER_W

    pltpu.sync_copy(uid_hbm.at[pl.ds(wid * IDX_ROWS, IDX_ROWS)], uidx)
    pltpu.sync_copy(iid_hbm.at[pl.ds(wid * IDX_ROWS, IDX_ROWS)], iidx)

    lanes = lax.iota(jnp.int32, 16)
    dnums = lax.GatherDimensionNumbers(
        offset_dims=(), collapsed_slice_dims=(0,), start_index_map=(0,))

    def id_at(idx_ref, n):
        vec = idx_ref[n // 128, pl.ds(((n % 128) // 16) * 16, 16)]
        pick = jnp.full((16,), n % 16, jnp.int32)
        b = lax.gather(vec, pick[:, None], dnums, (1,),
                       mode=lax.GatherScatterMode.PROMISE_IN_BOUNDS)
        return b[0]

    def issue(idx_ref, tab_hbm, ring, sem, n, k):
        idv = id_at(idx_ref, n)
        col = pl.multiple_of((idv // 128) * 128, 128)
        pltpu.async_copy(tab_hbm.at[:, pl.ds(col, 128)],
                         ring.at[:, pl.ds(k * 128, 128)], sem)

    def slot_wait(tab_hbm, ring, sem, k):
        pltpu.make_async_copy(tab_hbm.at[:, pl.ds(0, 128)],
                              ring.at[:, pl.ds(k * 128, 128)], sem).wait()

    def extract(idx_ref, ring, stage, n, k):
        idv = id_at(idx_ref, n)
        posv = jnp.full((16,), k * 128, jnp.int32) + (idv % 128)
        coln = jnp.full((16,), n, jnp.int32)
        lo = plsc.load_gather(ring, [lanes, posv])
        hi = plsc.load_gather(ring, [lanes + 16, posv])
        plsc.store_scatter(stage, [lanes, coln], lo)
        plsc.store_scatter(stage, [lanes + 16, coln], hi)

    for k in range(WIN):
        issue(uidx, ut_hbm, uring, usems[k], k, k)
        issue(iidx, it_hbm, iring, isems[k], k, k)

    def step(o, carry):
        for k in range(WIN):
            n = o * WIN + k
            slot_wait(ut_hbm, uring, usems[k], k)
            extract(uidx, uring, ustage, n, k)
            slot_wait(it_hbm, iring, isems[k], k)
            extract(iidx, iring, istage, n, k)

            @pl.when(n + WIN < B_PER_W)
            def _():
                issue(uidx, ut_hbm, uring, usems[k], n + WIN, k)
                issue(iidx, it_hbm, iring, isems[k], n + WIN, k)
        return carry

    lax.fori_loop(0, B_PER_W // WIN, step, 0)

    def group_body(g, carry):
        dot = jnp.zeros((16,), jnp.float32)
        nu = jnp.zeros((16,), jnp.float32)
        nv = jnp.zeros((16,), jnp.float32)
        for d in range(EMB_DIM):
            u = ustage[d, pl.ds(g * 16, 16)]
            v = istage[d, pl.ds(g * 16, 16)]
            dot = dot + u * v
            nu = nu + u * u
            nv = nv + v * v
        x = jnp.maximum(nu * nv, 1e-30)
        xi = lax.bitcast_convert_type(x, jnp.int32)
        y = lax.bitcast_convert_type(
            jnp.int32(0x5F3759DF) - (xi >> 1), jnp.float32)
        for _ in range(3):
            y = y * (1.5 - 0.5 * x * y * y)
        res[pl.ds(g * 16, 16)] = 6.0 * dot * y
        return carry

    lax.fori_loop(0, B_PER_W // 16, group_body, 0)

    pltpu.sync_copy(res, out_hbm.at[pl.ds(base, B_PER_W)])


def kernel(user_id, item_id, user_table, item_table):
    uid = user_id.astype(jnp.int32).reshape(128, 128)
    iid = item_id.astype(jnp.int32).reshape(128, 128)
    return _mf_sc_kernel(uid, iid, user_table.T, item_table.T)
